# baseline (speedup 1.0000x reference)
import functools

import jax
import jax.numpy as jnp
from jax.experimental import pallas as pl
from jax.experimental.pallas import tpu as pltpu
from jax.experimental.pallas import tpu_sc as plsc

B, N, D, C, K = 4, 4096, 1024, 2, 256

NB_A = 1024
NB_B = 2048


def _router_body(z_ref, Wc_ref, b1_ref, W2_ref, b2_ref, bh_ref,
                 scores_ref, seg_ref):
    zb = z_ref[...]
    combined = jax.lax.dot_general(zb, Wc_ref[...], (((1,), (0,)), ((), ())),
                                   preferred_element_type=jnp.float32)
    h = jnp.maximum(combined[:, :D] + b1_ref[...], 0.0)
    seg_ref[...] = combined[:, D:D + C] + bh_ref[...]
    s = jax.lax.dot_general(h, W2_ref[...], (((1,), (0,)), ((), ())),
                            preferred_element_type=jnp.float32) + b2_ref[...]
    scores_ref[...] = s


def _sortable_i32(bits):
    return jnp.where(bits < 0, bits ^ jnp.int32(0x7FFFFFFF), bits)


def _select_topk_mask(scores):
    kk = _sortable_i32(jax.lax.bitcast_convert_type(scores, jnp.int32))
    msb = jnp.int32(-2147483648)

    def step(i, prefix_u):
        bit = jnp.int32(1) << (jnp.int32(31) - i)
        cand_u = prefix_u | bit
        cand_s = cand_u ^ msb
        cnt = jnp.sum((kk >= cand_s).astype(jnp.int32), axis=1, keepdims=True)
        return jnp.where(cnt >= K, cand_u, prefix_u)

    prefix_u = jax.lax.fori_loop(0, 32, step, jnp.zeros((B, 1), jnp.int32))
    t_s = prefix_u ^ msb

    gt = kk > t_s
    eq = kk == t_s
    n_gt = jnp.sum(gt.astype(jnp.int32), axis=1, keepdims=True)
    need = K - n_gt

    col = jax.lax.broadcasted_iota(jnp.int32, (B, N), 1)
    fwd = jnp.int32(N) - col

    def step2(i, q):
        cand = q | (jnp.int32(1) << (jnp.int32(12) - i))
        cnt = jnp.sum((eq & (fwd >= cand)).astype(jnp.int32), axis=1,
                      keepdims=True)
        return jnp.where(cnt >= need, cand, q)

    q = jax.lax.fori_loop(0, 13, step2, jnp.zeros((B, 1), jnp.int32))
    sel_eq = eq & (fwd >= q)
    return gt | sel_eq


def _scal(x):
    return x if getattr(x, "ndim", 0) == 0 else jnp.max(x)


def _vreg_scan_desc(v, kk):
    rv = jax.lax.rev(v, (0,))
    c = plsc.cumsum(rv)
    lane = _scal(plsc.all_reduce_ffs(c >= kk))
    li = jax.lax.iota(jnp.int32, 16)
    n_above = _scal(jnp.sum(jnp.where(li < lane, rv, 0)))
    return jnp.int32(15) - lane, n_above


def _hist_scan_desc(h_ref, nvregs, kk):
    def body(j, carry):
        run, found, t_bin, n_above = carry
        vj = nvregs - 1 - j
        v = h_ref[pl.ds(vj * 16, 16)]
        s = _scal(jnp.sum(v))
        hit = jnp.logical_and(found == 0, run + s >= kk)
        bl, na = _vreg_scan_desc(v, kk - run)
        t_bin = jnp.where(hit, vj * 16 + bl, t_bin)
        n_above = jnp.where(hit, run + na, n_above)
        found = jnp.where(hit, jnp.int32(1), found)
        return run + s, found, t_bin, n_above
    init = (jnp.int32(0), jnp.int32(0), jnp.int32(0), jnp.int32(0))
    _, _, t_bin, n_above = jax.lax.fori_loop(0, nvregs, body, init)
    return t_bin, n_above


_NV = N // 16


def _sc_topk_body(scores_hbm, seg_hbm, w_hbm, clip_hbm,
                  s_v, u_v, h12_v, h8_v, segp_v, w_v, c16_v):
    cid = jax.lax.axis_index("c")
    sid = jax.lax.axis_index("s")
    wid = sid * 2 + cid

    @pl.when(wid < B)
    def _():
        b = wid
        pltpu.sync_copy(scores_hbm.at[pl.ds(pl.multiple_of(b * N, 8), N)], s_v)
        pltpu.sync_copy(
            seg_hbm.at[pl.ds(pl.multiple_of(b * N * C, 8), N * C)], segp_v)

        zeros16 = jnp.zeros((16,), jnp.int32)
        ones16 = jnp.ones((16,), jnp.int32)
        li = jax.lax.iota(jnp.int32, 16)
        msb = jnp.int32(-2147483648)

        def zero_hists(i, _):
            h12_v[pl.ds(i * 16, 16)] = zeros16
            return 0
        jax.lax.fori_loop(0, _NV, zero_hists, 0)

        def zero_h8(i, _):
            h8_v[pl.ds(i * 16, 16)] = zeros16
            return 0
        jax.lax.fori_loop(0, 16, zero_h8, 0)

        def keys(i, _):
            sv = s_v[pl.ds(i * 16, 16)]
            bits = jax.lax.bitcast_convert_type(sv, jnp.int32)
            ks = jnp.where(bits < 0, bits ^ jnp.int32(0x7FFFFFFF), bits)
            kx = ks ^ msb
            u_v[pl.ds(i * 16, 16)] = kx
            b12 = jax.lax.shift_right_logical(kx, 20)
            b8 = jax.lax.shift_right_logical(kx, 24)
            plsc.addupdate_scatter(h12_v, [b12], ones16)
            plsc.addupdate_scatter(h8_v, [b8], ones16)
            return 0
        jax.lax.fori_loop(0, _NV, keys, 0)

        t8, na8 = _hist_scan_desc(h8_v, 16, K)
        fine = h12_v[pl.ds(t8 * 16, 16)]
        bl, na12 = _vreg_scan_desc(fine, K - na8)
        T1 = t8 * 16 + bl
        na_1 = na8 + na12
        k2 = K - na_1

        jax.lax.fori_loop(0, _NV, zero_hists, 0)
        jax.lax.fori_loop(0, 16, zero_h8, 0)

        def hist2(i, _):
            kx = u_v[pl.ds(i * 16, 16)]
            mask = jax.lax.shift_right_logical(kx, 20) == T1
            m12 = jax.lax.shift_right_logical(kx, 8) & jnp.int32(0xFFF)
            m8 = jax.lax.shift_right_logical(kx, 12) & jnp.int32(0xFF)
            plsc.addupdate_scatter(h12_v, [m12], ones16, mask=mask)
            plsc.addupdate_scatter(h8_v, [m8], ones16, mask=mask)
            return 0
        jax.lax.fori_loop(0, _NV, hist2, 0)

        t8b, na8b = _hist_scan_desc(h8_v, 16, k2)
        fine2 = h12_v[pl.ds(t8b * 16, 16)]
        bl2, na12b = _vreg_scan_desc(fine2, k2 - na8b)
        T2 = t8b * 16 + bl2
        na_2 = na8b + na12b
        k3 = k2 - na_2

        jax.lax.fori_loop(0, 16, zero_h8, 0)

        def hist3(i, _):
            kx = u_v[pl.ds(i * 16, 16)]
            mask = jnp.logical_and(
                jax.lax.shift_right_logical(kx, 20) == T1,
                (jax.lax.shift_right_logical(kx, 8) & jnp.int32(0xFFF)) == T2)
            l8 = kx & jnp.int32(0xFF)
            plsc.addupdate_scatter(h8_v, [l8], ones16, mask=mask)
            return 0
        jax.lax.fori_loop(0, _NV, hist3, 0)

        T3, na_3 = _hist_scan_desc(h8_v, 16, k3)
        need = k3 - na_3

        inv_k = jnp.float32(1.0 / K)
        zf = jnp.zeros((16,), jnp.float32)

        def fin(i, ceq):
            kx = u_v[pl.ds(i * 16, 16)]
            b12 = jax.lax.shift_right_logical(kx, 20)
            m12 = jax.lax.shift_right_logical(kx, 8) & jnp.int32(0xFFF)
            l8 = kx & jnp.int32(0xFF)
            gt = jnp.logical_or(
                b12 > T1,
                jnp.logical_and(b12 == T1, jnp.logical_or(
                    m12 > T2, jnp.logical_and(m12 == T2, l8 > T3))))
            eq = jnp.logical_and(b12 == T1,
                                 jnp.logical_and(m12 == T2, l8 == T3))
            ec = plsc.cumsum(eq.astype(jnp.int32))
            sel = jnp.logical_or(gt, jnp.logical_and(eq, (ceq + ec) <= need))
            w_v[pl.ds(i * 16, 16)] = jnp.where(sel, inv_k, 0.0)
            return ceq + _scal(ec)
        jax.lax.fori_loop(0, _NV, fin, jnp.int32(0))

        half = jax.lax.shift_right_logical(li, 1)

        def red(j, acc):
            wexp = plsc.load_gather(w_v, [j * 8 + half])
            return acc + wexp * segp_v[pl.ds(j * 16, 16)]
        acc = jax.lax.fori_loop(0, N * C // 16, red, zf)

        even = (li & 1) == 0
        c0 = jnp.sum(jnp.where(even, acc, 0.0))
        c1 = jnp.sum(jnp.where(even, 0.0, acc))
        c16_v[...] = jnp.where(li == 0, c0, jnp.where(li == 1, c1, 0.0))

        pltpu.sync_copy(w_v, w_hbm.at[pl.ds(pl.multiple_of(b * N, 8), N)])
        pltpu.sync_copy(c16_v, clip_hbm.at[pl.ds(pl.multiple_of(b * 16, 8), 16)])


def _sc_topk(scores_flat, seg_flat):
    mesh = plsc.VectorSubcoreMesh(core_axis_name="c", subcore_axis_name="s")
    f = pl.kernel(
        _sc_topk_body, mesh=mesh,
        compiler_params=pltpu.CompilerParams(needs_layout_passes=False),
        out_type=[
            jax.ShapeDtypeStruct((B * N,), jnp.float32),
            jax.ShapeDtypeStruct((B * 16,), jnp.float32),
        ],
        scratch_types=[
            pltpu.VMEM((N,), jnp.float32),
            pltpu.VMEM((N,), jnp.int32),
            pltpu.VMEM((4096,), jnp.int32),
            pltpu.VMEM((256,), jnp.int32),
            pltpu.VMEM((N * C,), jnp.float32),
            pltpu.VMEM((N,), jnp.float32),
            pltpu.VMEM((16,), jnp.float32),
        ],
    )
    return f(scores_flat, seg_flat)


@jax.jit
def kernel(z, W1, b1, W2, b2, Wh, bh):
    z2d = z.reshape(B * N, D)
    Wc = jnp.concatenate([W1, Wh], axis=1)

    scores2d, seg2d = pl.pallas_call(
        _router_body,
        grid=(B * N // NB_A,),
        in_specs=[
            pl.BlockSpec((NB_A, D), lambda i: (i, 0)),
            pl.BlockSpec((D, D + C), lambda i: (0, 0)),
            pl.BlockSpec((1, D), lambda i: (0, 0)),
            pl.BlockSpec((D, 1), lambda i: (0, 0)),
            pl.BlockSpec((1, 1), lambda i: (0, 0)),
            pl.BlockSpec((1, C), lambda i: (0, 0)),
        ],
        out_specs=[
            pl.BlockSpec((NB_A, 1), lambda i: (i, 0)),
            pl.BlockSpec((NB_A, C), lambda i: (i, 0)),
        ],
        out_shape=[
            jax.ShapeDtypeStruct((B * N, 1), jnp.float32),
            jax.ShapeDtypeStruct((B * N, C), jnp.float32),
        ],
    )(z2d, Wc, b1.reshape(1, D), W2, b2.reshape(1, 1), bh.reshape(1, C))

    w_flat, clip_pad = _sc_topk(scores2d.reshape(B * N),
                                seg2d.reshape(B * N * C))
    clip_logits = clip_pad.reshape(B, 16)[:, :C]
    return clip_logits, seg2d.reshape(B, N, C), w_flat.reshape(B, N)


# speedup vs baseline: 1.0118x; 1.0118x over previous
import functools

import jax
import jax.numpy as jnp
from jax.experimental import pallas as pl
from jax.experimental.pallas import tpu as pltpu
from jax.experimental.pallas import tpu_sc as plsc

B, N, D, C, K = 4, 4096, 1024, 2, 256

NB_A = 1024
NB_B = 2048


def _router_body(z_ref, Wc_ref, b1_ref, W2_ref, b2_ref, bh_ref,
                 scores_ref, seg_ref):
    zb = z_ref[...]
    combined = jax.lax.dot_general(zb, Wc_ref[...], (((1,), (0,)), ((), ())),
                                   preferred_element_type=jnp.float32)
    h = jnp.maximum(combined[:, :D] + b1_ref[...], 0.0)
    seg_ref[...] = combined[:, D:D + C] + bh_ref[...]
    s = jax.lax.dot_general(h, W2_ref[...], (((1,), (0,)), ((), ())),
                            preferred_element_type=jnp.float32) + b2_ref[...]
    scores_ref[...] = s


def _sortable_i32(bits):
    return jnp.where(bits < 0, bits ^ jnp.int32(0x7FFFFFFF), bits)


def _select_topk_mask(scores):
    kk = _sortable_i32(jax.lax.bitcast_convert_type(scores, jnp.int32))
    msb = jnp.int32(-2147483648)

    def step(i, prefix_u):
        bit = jnp.int32(1) << (jnp.int32(31) - i)
        cand_u = prefix_u | bit
        cand_s = cand_u ^ msb
        cnt = jnp.sum((kk >= cand_s).astype(jnp.int32), axis=1, keepdims=True)
        return jnp.where(cnt >= K, cand_u, prefix_u)

    prefix_u = jax.lax.fori_loop(0, 32, step, jnp.zeros((B, 1), jnp.int32))
    t_s = prefix_u ^ msb

    gt = kk > t_s
    eq = kk == t_s
    n_gt = jnp.sum(gt.astype(jnp.int32), axis=1, keepdims=True)
    need = K - n_gt

    col = jax.lax.broadcasted_iota(jnp.int32, (B, N), 1)
    fwd = jnp.int32(N) - col

    def step2(i, q):
        cand = q | (jnp.int32(1) << (jnp.int32(12) - i))
        cnt = jnp.sum((eq & (fwd >= cand)).astype(jnp.int32), axis=1,
                      keepdims=True)
        return jnp.where(cnt >= need, cand, q)

    q = jax.lax.fori_loop(0, 13, step2, jnp.zeros((B, 1), jnp.int32))
    sel_eq = eq & (fwd >= q)
    return gt | sel_eq


def _scal(x):
    return x if getattr(x, "ndim", 0) == 0 else jnp.max(x)


def _vreg_scan_desc(v, kk):
    rv = jax.lax.rev(v, (0,))
    c = plsc.cumsum(rv)
    lane = _scal(plsc.all_reduce_ffs(c >= kk))
    li = jax.lax.iota(jnp.int32, 16)
    n_above = _scal(jnp.sum(jnp.where(li < lane, rv, 0)))
    return jnp.int32(15) - lane, n_above


def _hist_scan_desc(h_ref, nvregs, kk):
    def body(j, carry):
        run, found, t_bin, n_above = carry
        vj = nvregs - 1 - j
        v = h_ref[pl.ds(vj * 16, 16)]
        s = _scal(jnp.sum(v))
        hit = jnp.logical_and(found == 0, run + s >= kk)
        bl, na = _vreg_scan_desc(v, kk - run)
        t_bin = jnp.where(hit, vj * 16 + bl, t_bin)
        n_above = jnp.where(hit, run + na, n_above)
        found = jnp.where(hit, jnp.int32(1), found)
        return run + s, found, t_bin, n_above
    init = (jnp.int32(0), jnp.int32(0), jnp.int32(0), jnp.int32(0))
    _, _, t_bin, n_above = jax.lax.fori_loop(0, nvregs, body, init)
    return t_bin, n_above


_NV = N // 16


def _sc_topk_body(scores_hbm, seg_hbm, w_hbm, clip_hbm,
                  s_v, u_v, h12_v, h8_v, segp_v, w_v, c16_v):
    cid = jax.lax.axis_index("c")
    sid = jax.lax.axis_index("s")
    wid = sid * 2 + cid

    @pl.when(wid < B)
    def _():
        b = wid
        pltpu.sync_copy(scores_hbm.at[pl.ds(pl.multiple_of(b * N, 8), N)], s_v)
        pltpu.sync_copy(
            seg_hbm.at[pl.ds(pl.multiple_of(b * N * C, 8), N * C)], segp_v)

        zeros16 = jnp.zeros((16,), jnp.int32)
        ones16 = jnp.ones((16,), jnp.int32)
        li = jax.lax.iota(jnp.int32, 16)
        msb = jnp.int32(-2147483648)

        def zero_hists(i, _):
            h12_v[pl.ds(i * 16, 16)] = zeros16
            return 0
        jax.lax.fori_loop(0, _NV, zero_hists, 0, unroll=8)

        def zero_h8(i, _):
            h8_v[pl.ds(i * 16, 16)] = zeros16
            return 0
        jax.lax.fori_loop(0, 16, zero_h8, 0, unroll=8)

        def keys(i, _):
            sv = s_v[pl.ds(i * 16, 16)]
            bits = jax.lax.bitcast_convert_type(sv, jnp.int32)
            ks = jnp.where(bits < 0, bits ^ jnp.int32(0x7FFFFFFF), bits)
            kx = ks ^ msb
            u_v[pl.ds(i * 16, 16)] = kx
            b12 = jax.lax.shift_right_logical(kx, 20)
            b8 = jax.lax.shift_right_logical(kx, 24)
            plsc.addupdate_scatter(h12_v, [b12], ones16)
            plsc.addupdate_scatter(h8_v, [b8], ones16)
            return 0
        jax.lax.fori_loop(0, _NV, keys, 0, unroll=8)

        t8, na8 = _hist_scan_desc(h8_v, 16, K)
        fine = h12_v[pl.ds(t8 * 16, 16)]
        bl, na12 = _vreg_scan_desc(fine, K - na8)
        T1 = t8 * 16 + bl
        na_1 = na8 + na12
        k2 = K - na_1

        jax.lax.fori_loop(0, _NV, zero_hists, 0)
        jax.lax.fori_loop(0, 16, zero_h8, 0)

        def hist2(i, _):
            kx = u_v[pl.ds(i * 16, 16)]
            mask = jax.lax.shift_right_logical(kx, 20) == T1
            m12 = jax.lax.shift_right_logical(kx, 8) & jnp.int32(0xFFF)
            m8 = jax.lax.shift_right_logical(kx, 12) & jnp.int32(0xFF)
            plsc.addupdate_scatter(h12_v, [m12], ones16, mask=mask)
            plsc.addupdate_scatter(h8_v, [m8], ones16, mask=mask)
            return 0
        jax.lax.fori_loop(0, _NV, hist2, 0, unroll=8)

        t8b, na8b = _hist_scan_desc(h8_v, 16, k2)
        fine2 = h12_v[pl.ds(t8b * 16, 16)]
        bl2, na12b = _vreg_scan_desc(fine2, k2 - na8b)
        T2 = t8b * 16 + bl2
        na_2 = na8b + na12b
        k3 = k2 - na_2

        jax.lax.fori_loop(0, 16, zero_h8, 0)

        def hist3(i, _):
            kx = u_v[pl.ds(i * 16, 16)]
            mask = jnp.logical_and(
                jax.lax.shift_right_logical(kx, 20) == T1,
                (jax.lax.shift_right_logical(kx, 8) & jnp.int32(0xFFF)) == T2)
            l8 = kx & jnp.int32(0xFF)
            plsc.addupdate_scatter(h8_v, [l8], ones16, mask=mask)
            return 0
        jax.lax.fori_loop(0, _NV, hist3, 0, unroll=8)

        T3, na_3 = _hist_scan_desc(h8_v, 16, k3)
        need = k3 - na_3

        inv_k = jnp.float32(1.0 / K)
        zf = jnp.zeros((16,), jnp.float32)
        need_v = jnp.full((16,), 1, jnp.int32) * need
        half = jax.lax.shift_right_logical(li, 1)

        def fin(i, carry):
            ceq_v, acc = carry
            kx = u_v[pl.ds(i * 16, 16)]
            b12 = jax.lax.shift_right_logical(kx, 20)
            m12 = jax.lax.shift_right_logical(kx, 8) & jnp.int32(0xFFF)
            l8 = kx & jnp.int32(0xFF)
            gt = jnp.logical_or(
                b12 > T1,
                jnp.logical_and(b12 == T1, jnp.logical_or(
                    m12 > T2, jnp.logical_and(m12 == T2, l8 > T3))))
            eq = jnp.logical_and(b12 == T1,
                                 jnp.logical_and(m12 == T2, l8 == T3))
            ec = plsc.cumsum(eq.astype(jnp.int32))
            sel = jnp.logical_or(
                gt, jnp.logical_and(eq, (ceq_v + ec) <= need_v))
            w_v[pl.ds(i * 16, 16)] = jnp.where(sel, inv_k, 0.0)
            ceq_v = ceq_v + plsc.all_reduce_population_count(eq)
            wexp0 = plsc.load_gather(w_v, [i * 16 + half])
            wexp1 = plsc.load_gather(w_v, [i * 16 + 8 + half])
            acc = acc + wexp0 * segp_v[pl.ds(i * 32, 16)]
            acc = acc + wexp1 * segp_v[pl.ds(i * 32 + 16, 16)]
            return ceq_v, acc
        _, acc = jax.lax.fori_loop(
            0, _NV, fin, (jnp.zeros((16,), jnp.int32), zf), unroll=4)

        even = (li & 1) == 0
        c0 = jnp.sum(jnp.where(even, acc, 0.0))
        c1 = jnp.sum(jnp.where(even, 0.0, acc))
        c16_v[...] = jnp.where(li == 0, c0, jnp.where(li == 1, c1, 0.0))

        pltpu.sync_copy(w_v, w_hbm.at[pl.ds(pl.multiple_of(b * N, 8), N)])
        pltpu.sync_copy(c16_v, clip_hbm.at[pl.ds(pl.multiple_of(b * 16, 8), 16)])


def _sc_topk(scores_flat, seg_flat):
    mesh = plsc.VectorSubcoreMesh(core_axis_name="c", subcore_axis_name="s")
    f = pl.kernel(
        _sc_topk_body, mesh=mesh,
        compiler_params=pltpu.CompilerParams(needs_layout_passes=False),
        out_type=[
            jax.ShapeDtypeStruct((B * N,), jnp.float32),
            jax.ShapeDtypeStruct((B * 16,), jnp.float32),
        ],
        scratch_types=[
            pltpu.VMEM((N,), jnp.float32),
            pltpu.VMEM((N,), jnp.int32),
            pltpu.VMEM((4096,), jnp.int32),
            pltpu.VMEM((256,), jnp.int32),
            pltpu.VMEM((N * C,), jnp.float32),
            pltpu.VMEM((N,), jnp.float32),
            pltpu.VMEM((16,), jnp.float32),
        ],
    )
    return f(scores_flat, seg_flat)


@jax.jit
def kernel(z, W1, b1, W2, b2, Wh, bh):
    z2d = z.reshape(B * N, D)
    Wc = jnp.concatenate([W1, Wh], axis=1)

    scores2d, seg2d = pl.pallas_call(
        _router_body,
        grid=(B * N // NB_A,),
        in_specs=[
            pl.BlockSpec((NB_A, D), lambda i: (i, 0)),
            pl.BlockSpec((D, D + C), lambda i: (0, 0)),
            pl.BlockSpec((1, D), lambda i: (0, 0)),
            pl.BlockSpec((D, 1), lambda i: (0, 0)),
            pl.BlockSpec((1, 1), lambda i: (0, 0)),
            pl.BlockSpec((1, C), lambda i: (0, 0)),
        ],
        out_specs=[
            pl.BlockSpec((NB_A, 1), lambda i: (i, 0)),
            pl.BlockSpec((NB_A, C), lambda i: (i, 0)),
        ],
        out_shape=[
            jax.ShapeDtypeStruct((B * N, 1), jnp.float32),
            jax.ShapeDtypeStruct((B * N, C), jnp.float32),
        ],
    )(z2d, Wc, b1.reshape(1, D), W2, b2.reshape(1, 1), bh.reshape(1, C))

    w_flat, clip_pad = _sc_topk(scores2d.reshape(B * N),
                                seg2d.reshape(B * N * C))
    clip_logits = clip_pad.reshape(B, 16)[:, :C]
    return clip_logits, seg2d.reshape(B, N, C), w_flat.reshape(B, N)


# speedup vs baseline: 1.0363x; 1.0242x over previous
import functools

import jax
import jax.numpy as jnp
from jax.experimental import pallas as pl
from jax.experimental.pallas import tpu as pltpu
from jax.experimental.pallas import tpu_sc as plsc

B, N, D, C, K = 4, 4096, 1024, 2, 256

NB_A = 1024
NB_B = 2048


def _router_body(z_ref, Wc_ref, b1_ref, W2_ref, b2_ref, bh_ref,
                 scores_ref, seg_ref):
    zb = z_ref[...]
    combined = jax.lax.dot_general(zb, Wc_ref[...], (((1,), (0,)), ((), ())),
                                   preferred_element_type=jnp.float32)
    h = jnp.maximum(combined[:, :D] + b1_ref[...], 0.0)
    seg_ref[...] = combined[:, D:D + C] + bh_ref[...]
    s = jax.lax.dot_general(h, W2_ref[...], (((1,), (0,)), ((), ())),
                            preferred_element_type=jnp.float32) + b2_ref[...]
    scores_ref[...] = s


def _sortable_i32(bits):
    return jnp.where(bits < 0, bits ^ jnp.int32(0x7FFFFFFF), bits)


def _select_topk_mask(scores):
    kk = _sortable_i32(jax.lax.bitcast_convert_type(scores, jnp.int32))
    msb = jnp.int32(-2147483648)

    def step(i, prefix_u):
        bit = jnp.int32(1) << (jnp.int32(31) - i)
        cand_u = prefix_u | bit
        cand_s = cand_u ^ msb
        cnt = jnp.sum((kk >= cand_s).astype(jnp.int32), axis=1, keepdims=True)
        return jnp.where(cnt >= K, cand_u, prefix_u)

    prefix_u = jax.lax.fori_loop(0, 32, step, jnp.zeros((B, 1), jnp.int32))
    t_s = prefix_u ^ msb

    gt = kk > t_s
    eq = kk == t_s
    n_gt = jnp.sum(gt.astype(jnp.int32), axis=1, keepdims=True)
    need = K - n_gt

    col = jax.lax.broadcasted_iota(jnp.int32, (B, N), 1)
    fwd = jnp.int32(N) - col

    def step2(i, q):
        cand = q | (jnp.int32(1) << (jnp.int32(12) - i))
        cnt = jnp.sum((eq & (fwd >= cand)).astype(jnp.int32), axis=1,
                      keepdims=True)
        return jnp.where(cnt >= need, cand, q)

    q = jax.lax.fori_loop(0, 13, step2, jnp.zeros((B, 1), jnp.int32))
    sel_eq = eq & (fwd >= q)
    return gt | sel_eq


def _scal(x):
    return x if getattr(x, "ndim", 0) == 0 else jnp.max(x)


def _vreg_scan_desc(v, kk):
    rv = jax.lax.rev(v, (0,))
    c = plsc.cumsum(rv)
    lane = _scal(plsc.all_reduce_ffs(c >= kk))
    li = jax.lax.iota(jnp.int32, 16)
    n_above = _scal(jnp.sum(jnp.where(li < lane, rv, 0)))
    return jnp.int32(15) - lane, n_above


def _hist_scan_desc(h_ref, nvregs, kk):
    def body(j, carry):
        run, found, t_bin, n_above = carry
        vj = nvregs - 1 - j
        v = h_ref[pl.ds(vj * 16, 16)]
        s = _scal(jnp.sum(v))
        hit = jnp.logical_and(found == 0, run + s >= kk)
        bl, na = _vreg_scan_desc(v, kk - run)
        t_bin = jnp.where(hit, vj * 16 + bl, t_bin)
        n_above = jnp.where(hit, run + na, n_above)
        found = jnp.where(hit, jnp.int32(1), found)
        return run + s, found, t_bin, n_above
    init = (jnp.int32(0), jnp.int32(0), jnp.int32(0), jnp.int32(0))
    _, _, t_bin, n_above = jax.lax.fori_loop(0, nvregs, body, init)
    return t_bin, n_above


_NV = N // 16


def _thresh_body(scores_ref, tn_ref):
    kk = _sortable_i32(jax.lax.bitcast_convert_type(scores_ref[...], jnp.int32))
    msb = jnp.int32(-2147483648)

    def step(i, prefix_u):
        cand_u = prefix_u | (jnp.int32(1) << (jnp.int32(31) - i))
        cand_s = cand_u ^ msb
        cnt = jnp.sum((kk >= cand_s).astype(jnp.int32), axis=1, keepdims=True)
        return jnp.where(cnt >= K, cand_u, prefix_u)

    prefix_u = jax.lax.fori_loop(0, 32, step, jnp.zeros((B, 1), jnp.int32))
    t_s = prefix_u ^ msb
    n_gt = jnp.sum((kk > t_s).astype(jnp.int32), axis=1, keepdims=True)
    need = K - n_gt
    tn_ref[...] = jnp.concatenate(
        [jnp.broadcast_to(t_s, (B, 16)), jnp.broadcast_to(need, (B, 16))],
        axis=1)


def _sc_apply_body(scores_hbm, seg_hbm, tn_hbm, w_hbm, clip_hbm,
                   s_v, segp_v, tn_v, w_v, c16_v):
    cid = jax.lax.axis_index("c")
    sid = jax.lax.axis_index("s")
    wid = sid * 2 + cid

    @pl.when(wid < B)
    def _():
        b = wid
        pltpu.sync_copy(scores_hbm.at[pl.ds(pl.multiple_of(b * N, 8), N)], s_v)
        pltpu.sync_copy(
            seg_hbm.at[pl.ds(pl.multiple_of(b * N * C, 8), N * C)], segp_v)
        pltpu.sync_copy(tn_hbm.at[pl.ds(pl.multiple_of(b * 32, 8), 32)], tn_v)

        li = jax.lax.iota(jnp.int32, 16)
        t_vec = tn_v[pl.ds(0, 16)]
        need_v = tn_v[pl.ds(16, 16)]
        inv_k = jnp.float32(1.0 / K)
        zf = jnp.zeros((16,), jnp.float32)
        half = jax.lax.shift_right_logical(li, 1)

        def fin(i, carry):
            ceq_v, acc = carry
            sv = s_v[pl.ds(i * 16, 16)]
            bits = jax.lax.bitcast_convert_type(sv, jnp.int32)
            ks = jnp.where(bits < 0, bits ^ jnp.int32(0x7FFFFFFF), bits)
            gt = ks > t_vec
            eq = ks == t_vec
            ec = plsc.cumsum(eq.astype(jnp.int32))
            sel = jnp.logical_or(
                gt, jnp.logical_and(eq, (ceq_v + ec) <= need_v))
            w_v[pl.ds(i * 16, 16)] = jnp.where(sel, inv_k, 0.0)
            ceq_v = ceq_v + plsc.all_reduce_population_count(eq)
            wexp0 = plsc.load_gather(w_v, [i * 16 + half])
            wexp1 = plsc.load_gather(w_v, [i * 16 + 8 + half])
            acc = acc + wexp0 * segp_v[pl.ds(i * 32, 16)]
            acc = acc + wexp1 * segp_v[pl.ds(i * 32 + 16, 16)]
            return ceq_v, acc
        _, acc = jax.lax.fori_loop(
            0, _NV, fin, (jnp.zeros((16,), jnp.int32), zf), unroll=8)

        even = (li & 1) == 0
        c0 = jnp.sum(jnp.where(even, acc, 0.0))
        c1 = jnp.sum(jnp.where(even, 0.0, acc))
        c16_v[...] = jnp.where(li == 0, c0, jnp.where(li == 1, c1, 0.0))

        pltpu.sync_copy(w_v, w_hbm.at[pl.ds(pl.multiple_of(b * N, 8), N)])
        pltpu.sync_copy(c16_v, clip_hbm.at[pl.ds(pl.multiple_of(b * 16, 8), 16)])


def _sc_apply(scores_flat, seg_flat, tn_flat):
    mesh = plsc.VectorSubcoreMesh(core_axis_name="c", subcore_axis_name="s")
    f = pl.kernel(
        _sc_apply_body, mesh=mesh,
        compiler_params=pltpu.CompilerParams(needs_layout_passes=False),
        out_type=[
            jax.ShapeDtypeStruct((B * N,), jnp.float32),
            jax.ShapeDtypeStruct((B * 16,), jnp.float32),
        ],
        scratch_types=[
            pltpu.VMEM((N,), jnp.float32),
            pltpu.VMEM((N * C,), jnp.float32),
            pltpu.VMEM((32,), jnp.int32),
            pltpu.VMEM((N,), jnp.float32),
            pltpu.VMEM((16,), jnp.float32),
        ],
    )
    return f(scores_flat, seg_flat, tn_flat)


@jax.jit
def kernel(z, W1, b1, W2, b2, Wh, bh):
    z2d = z.reshape(B * N, D)
    Wc = jnp.concatenate([W1, Wh], axis=1)

    scores2d, seg2d = pl.pallas_call(
        _router_body,
        grid=(B * N // NB_A,),
        in_specs=[
            pl.BlockSpec((NB_A, D), lambda i: (i, 0)),
            pl.BlockSpec((D, D + C), lambda i: (0, 0)),
            pl.BlockSpec((1, D), lambda i: (0, 0)),
            pl.BlockSpec((D, 1), lambda i: (0, 0)),
            pl.BlockSpec((1, 1), lambda i: (0, 0)),
            pl.BlockSpec((1, C), lambda i: (0, 0)),
        ],
        out_specs=[
            pl.BlockSpec((NB_A, 1), lambda i: (i, 0)),
            pl.BlockSpec((NB_A, C), lambda i: (i, 0)),
        ],
        out_shape=[
            jax.ShapeDtypeStruct((B * N, 1), jnp.float32),
            jax.ShapeDtypeStruct((B * N, C), jnp.float32),
        ],
    )(z2d, Wc, b1.reshape(1, D), W2, b2.reshape(1, 1), bh.reshape(1, C))

    tn = pl.pallas_call(
        _thresh_body,
        in_specs=[pl.BlockSpec((B, N), lambda: (0, 0))],
        out_specs=pl.BlockSpec((B, 32), lambda: (0, 0)),
        out_shape=jax.ShapeDtypeStruct((B, 32), jnp.int32),
    )(scores2d.reshape(B, N))

    w_flat, clip_pad = _sc_apply(scores2d.reshape(B * N),
                                 seg2d.reshape(B * N * C),
                                 tn.reshape(B * 32))
    clip_logits = clip_pad.reshape(B, 16)[:, :C]
    return clip_logits, seg2d.reshape(B, N, C), w_flat.reshape(B, N)
